# Initial kernel scaffold; baseline (speedup 1.0000x reference)
import jax
import jax.numpy as jnp
from jax.experimental import pallas as pl


def kernel(all_embeddings, assoc, src, pos_dst, neg_dst_surv, neg_src_surv, W_omega, b_omega, psi):
    raise NotImplementedError("write your pallas kernel here")


# speedup vs baseline: 44.6896x; 44.6896x over previous
import functools

import jax
import jax.numpy as jnp
from jax import lax
from jax.experimental import pallas as pl
from jax.experimental.pallas import tpu as pltpu
from jax.experimental.pallas import tpu_sc as plsc

EMBED_DIM = 512
NUM_SURV = 10
N_NODES = 10000
BATCH = 8192
_ROWS_PER_BLK = 1000


def _pq_body(w_ref, e_ref, p_ref, q_ref):
    e = e_ref[...]
    wu = w_ref[:, :EMBED_DIM]
    wv = w_ref[:, EMBED_DIM:]
    p_ref[...] = jnp.sum(e * wu, axis=1, keepdims=True)
    q_ref[...] = jnp.sum(e * wv, axis=1, keepdims=True)


def _compute_pq(all_embeddings, w_omega):
    nblk = N_NODES // _ROWS_PER_BLK
    p, q = pl.pallas_call(
        _pq_body,
        grid=(nblk,),
        in_specs=[
            pl.BlockSpec((1, 2 * EMBED_DIM), lambda i: (0, 0)),
            pl.BlockSpec((_ROWS_PER_BLK, EMBED_DIM), lambda i: (i, 0)),
        ],
        out_specs=[
            pl.BlockSpec((_ROWS_PER_BLK, 1), lambda i: (i, 0)),
            pl.BlockSpec((_ROWS_PER_BLK, 1), lambda i: (i, 0)),
        ],
        out_shape=[
            jax.ShapeDtypeStruct((N_NODES, 1), jnp.float32),
            jax.ShapeDtypeStruct((N_NODES, 1), jnp.float32),
        ],
    )(w_omega, all_embeddings)
    return p.reshape(N_NODES), q.reshape(N_NODES)


def _sc_logits(assoc, p, q, src, pos_dst, neg_dst_surv, neg_src_surv):
    mesh = plsc.VectorSubcoreMesh(core_axis_name="c", subcore_axis_name="s")
    nw = mesh.num_cores * mesh.num_subcores
    nb = BATCH // nw
    ns = (BATCH * NUM_SURV) // nw
    num_cores = mesh.num_cores
    rep_idx = jnp.arange(ns, dtype=jnp.int32) // NUM_SURV

    @functools.partial(
        pl.kernel,
        out_type=[
            jax.ShapeDtypeStruct((BATCH,), jnp.float32),
            jax.ShapeDtypeStruct((BATCH * NUM_SURV,), jnp.float32),
            jax.ShapeDtypeStruct((BATCH * NUM_SURV,), jnp.float32),
        ],
        mesh=mesh,
        compiler_params=pltpu.CompilerParams(needs_layout_passes=False),
        scratch_types=[
            pltpu.VMEM((N_NODES,), jnp.int32),
            pltpu.VMEM((N_NODES,), jnp.float32),
            pltpu.VMEM((N_NODES,), jnp.float32),
            pltpu.VMEM((nb,), jnp.int32),
            pltpu.VMEM((nb,), jnp.int32),
            pltpu.VMEM((ns,), jnp.int32),
            pltpu.VMEM((ns,), jnp.int32),
            pltpu.VMEM((ns,), jnp.int32),
            pltpu.VMEM((nb,), jnp.float32),
            pltpu.VMEM((nb,), jnp.float32),
            pltpu.VMEM((nb,), jnp.float32),
            pltpu.VMEM((ns,), jnp.float32),
            pltpu.VMEM((ns,), jnp.float32),
        ],
    )
    def k(assoc_h, p_h, q_h, src_h, pos_h, negd_h, negs_h, rep_h,
          olam_h, osu_h, osv_h,
          assoc_v, p_v, q_v, src_v, pos_v, negd_v, negs_v, rep_v,
          psrc_v, qdst_v, lam_v, su_v, sv_v):
        wid = lax.axis_index("s") * num_cores + lax.axis_index("c")
        pltpu.sync_copy(assoc_h, assoc_v)
        pltpu.sync_copy(p_h, p_v)
        pltpu.sync_copy(q_h, q_v)
        pltpu.sync_copy(src_h.at[pl.ds(wid * nb, nb)], src_v)
        pltpu.sync_copy(pos_h.at[pl.ds(wid * nb, nb)], pos_v)
        pltpu.sync_copy(negd_h.at[pl.ds(wid * ns, ns)], negd_v)
        pltpu.sync_copy(negs_h.at[pl.ds(wid * ns, ns)], negs_v)
        pltpu.sync_copy(rep_h, rep_v)

        def lam_body(t, carry):
            sl = pl.ds(t * 16, 16)
            pe = plsc.load_gather(p_v, [plsc.load_gather(assoc_v, [src_v[sl]])])
            qe = plsc.load_gather(q_v, [plsc.load_gather(assoc_v, [pos_v[sl]])])
            psrc_v[sl] = pe
            qdst_v[sl] = qe
            lam_v[sl] = pe + qe
            return carry

        lax.fori_loop(0, nb // 16, lam_body, 0)

        def surv_body(t, carry):
            sl = pl.ds(t * 16, 16)
            ri = rep_v[sl]
            pe = plsc.load_gather(psrc_v, [ri])
            qe = plsc.load_gather(q_v, [plsc.load_gather(assoc_v, [negd_v[sl]])])
            su_v[sl] = pe + qe
            pe2 = plsc.load_gather(p_v, [plsc.load_gather(assoc_v, [negs_v[sl]])])
            qe2 = plsc.load_gather(qdst_v, [ri])
            sv_v[sl] = pe2 + qe2
            return carry

        lax.fori_loop(0, ns // 16, surv_body, 0)

        pltpu.sync_copy(lam_v, olam_h.at[pl.ds(wid * nb, nb)])
        pltpu.sync_copy(su_v, osu_h.at[pl.ds(wid * ns, ns)])
        pltpu.sync_copy(sv_v, osv_h.at[pl.ds(wid * ns, ns)])

    return k(assoc, p, q, src, pos_dst, neg_dst_surv, neg_src_surv, rep_idx)


def _loss_body(b_ref, psi_ref, slam_ref, ssu_ref, ssv_ref, o1, o2, o3):
    b = b_ref[0]
    psi = psi_ref[0]
    pe = psi + 1e-7
    gl = (slam_ref[...] + b) / pe
    lam = psi * (jnp.log(1.0 + jnp.exp(-gl)) + gl)
    o1[0] = -jnp.sum(jnp.log(lam + 1e-10)) / BATCH
    gu = (ssu_ref[...] + b) / pe
    su = psi * (jnp.log(1.0 + jnp.exp(-gu)) + gu)
    o2[0] = jnp.sum(su) / NUM_SURV / BATCH
    gv = (ssv_ref[...] + b) / pe
    sv = psi * (jnp.log(1.0 + jnp.exp(-gv)) + gv)
    o3[0] = jnp.sum(sv) / NUM_SURV / BATCH


def _losses(s_lam, s_su, s_sv, b_omega, psi):
    o1, o2, o3 = pl.pallas_call(
        _loss_body,
        in_specs=[
            pl.BlockSpec(memory_space=pltpu.SMEM),
            pl.BlockSpec(memory_space=pltpu.SMEM),
            pl.BlockSpec((BATCH // 128, 128), lambda: (0, 0)),
            pl.BlockSpec((BATCH * NUM_SURV // 128, 128), lambda: (0, 0)),
            pl.BlockSpec((BATCH * NUM_SURV // 128, 128), lambda: (0, 0)),
        ],
        out_specs=[
            pl.BlockSpec(memory_space=pltpu.SMEM),
            pl.BlockSpec(memory_space=pltpu.SMEM),
            pl.BlockSpec(memory_space=pltpu.SMEM),
        ],
        out_shape=[jax.ShapeDtypeStruct((1,), jnp.float32)] * 3,
    )(
        b_omega, psi,
        s_lam.reshape(BATCH // 128, 128),
        s_su.reshape(BATCH * NUM_SURV // 128, 128),
        s_sv.reshape(BATCH * NUM_SURV // 128, 128),
    )
    return o1[0], o2[0], o3[0]


def kernel(all_embeddings, assoc, src, pos_dst, neg_dst_surv, neg_src_surv,
           W_omega, b_omega, psi):
    assoc_i = assoc.astype(jnp.int32)
    p, q = _compute_pq(all_embeddings, W_omega)
    s_lam, s_su, s_sv = _sc_logits(
        assoc_i, p, q, src, pos_dst, neg_dst_surv, neg_src_surv)
    return _losses(s_lam, s_su, s_sv, b_omega, psi)


# speedup vs baseline: 55.1712x; 1.2345x over previous
import functools

import jax
import jax.numpy as jnp
from jax import lax
from jax.experimental import pallas as pl
from jax.experimental.pallas import tpu as pltpu
from jax.experimental.pallas import tpu_sc as plsc

EMBED_DIM = 512
NUM_SURV = 10
N_NODES = 10000
BATCH = 8192
_ROWS_PER_BLK = 1000


def _pq_body(w_ref, e_ref, p_ref, q_ref):
    e = e_ref[...]
    wu = w_ref[:, :EMBED_DIM]
    wv = w_ref[:, EMBED_DIM:]
    dn = (((1,), (1,)), ((), ()))
    pt = lax.dot_general(wu, e, dn, preferred_element_type=jnp.float32)
    qt = lax.dot_general(wv, e, dn, preferred_element_type=jnp.float32)
    p_ref[...] = pt.reshape(N_NODES)
    q_ref[...] = qt.reshape(N_NODES)


def _compute_pq(all_embeddings, w_omega):
    p, q = pl.pallas_call(
        _pq_body,
        out_shape=[
            jax.ShapeDtypeStruct((N_NODES,), jnp.float32),
            jax.ShapeDtypeStruct((N_NODES,), jnp.float32),
        ],
    )(w_omega, all_embeddings)
    return p, q


def _sc_logits(assoc, p, q, src, pos_dst, neg_dst_surv, neg_src_surv):
    mesh = plsc.VectorSubcoreMesh(core_axis_name="c", subcore_axis_name="s")
    nw = mesh.num_cores * mesh.num_subcores
    nb = BATCH // nw
    ns = (BATCH * NUM_SURV) // nw
    num_cores = mesh.num_cores
    rep_idx = jnp.arange(ns, dtype=jnp.int32) // NUM_SURV

    @functools.partial(
        pl.kernel,
        out_type=[
            jax.ShapeDtypeStruct((BATCH,), jnp.float32),
            jax.ShapeDtypeStruct((BATCH * NUM_SURV,), jnp.float32),
            jax.ShapeDtypeStruct((BATCH * NUM_SURV,), jnp.float32),
        ],
        mesh=mesh,
        compiler_params=pltpu.CompilerParams(needs_layout_passes=False),
        scratch_types=[
            pltpu.VMEM((N_NODES,), jnp.int32),
            pltpu.VMEM((N_NODES,), jnp.float32),
            pltpu.VMEM((N_NODES,), jnp.float32),
            pltpu.VMEM((nb,), jnp.int32),
            pltpu.VMEM((nb,), jnp.int32),
            pltpu.VMEM((ns,), jnp.int32),
            pltpu.VMEM((ns,), jnp.int32),
            pltpu.VMEM((ns,), jnp.int32),
            pltpu.VMEM((nb,), jnp.float32),
            pltpu.VMEM((nb,), jnp.float32),
            pltpu.VMEM((nb,), jnp.float32),
            pltpu.VMEM((ns,), jnp.float32),
            pltpu.VMEM((ns,), jnp.float32),
        ],
    )
    def k(assoc_h, p_h, q_h, src_h, pos_h, negd_h, negs_h, rep_h,
          olam_h, osu_h, osv_h,
          assoc_v, p_v, q_v, src_v, pos_v, negd_v, negs_v, rep_v,
          psrc_v, qdst_v, lam_v, su_v, sv_v):
        wid = lax.axis_index("s") * num_cores + lax.axis_index("c")
        pltpu.sync_copy(assoc_h, assoc_v)
        pltpu.sync_copy(p_h, p_v)
        pltpu.sync_copy(q_h, q_v)
        pltpu.sync_copy(src_h.at[pl.ds(wid * nb, nb)], src_v)
        pltpu.sync_copy(pos_h.at[pl.ds(wid * nb, nb)], pos_v)
        pltpu.sync_copy(negd_h.at[pl.ds(wid * ns, ns)], negd_v)
        pltpu.sync_copy(negs_h.at[pl.ds(wid * ns, ns)], negs_v)
        pltpu.sync_copy(rep_h, rep_v)

        def lam_body(t, carry):
            for u in range(2):
                sl = pl.ds(t * 32 + u * 16, 16)
                pe = plsc.load_gather(
                    p_v, [plsc.load_gather(assoc_v, [src_v[sl]])])
                qe = plsc.load_gather(
                    q_v, [plsc.load_gather(assoc_v, [pos_v[sl]])])
                psrc_v[sl] = pe
                qdst_v[sl] = qe
                lam_v[sl] = pe + qe
            return carry

        lax.fori_loop(0, nb // 32, lam_body, 0)

        def surv_body(t, carry):
            for u in range(4):
                sl = pl.ds(t * 64 + u * 16, 16)
                ri = rep_v[sl]
                pe = plsc.load_gather(psrc_v, [ri])
                qe = plsc.load_gather(
                    q_v, [plsc.load_gather(assoc_v, [negd_v[sl]])])
                su_v[sl] = pe + qe
                pe2 = plsc.load_gather(
                    p_v, [plsc.load_gather(assoc_v, [negs_v[sl]])])
                qe2 = plsc.load_gather(qdst_v, [ri])
                sv_v[sl] = pe2 + qe2
            return carry

        lax.fori_loop(0, ns // 64, surv_body, 0)

        pltpu.sync_copy(lam_v, olam_h.at[pl.ds(wid * nb, nb)])
        pltpu.sync_copy(su_v, osu_h.at[pl.ds(wid * ns, ns)])
        pltpu.sync_copy(sv_v, osv_h.at[pl.ds(wid * ns, ns)])

    return k(assoc, p, q, src, pos_dst, neg_dst_surv, neg_src_surv, rep_idx)


def _loss_body(b_ref, psi_ref, slam_ref, ssu_ref, ssv_ref, o1, o2, o3):
    b = b_ref[0]
    psi = psi_ref[0]
    pe = psi + 1e-7
    gl = (slam_ref[...] + b) / pe
    lam = psi * (jnp.log(1.0 + jnp.exp(-gl)) + gl)
    o1[0] = -jnp.sum(jnp.log(lam + 1e-10)) / BATCH
    gu = (ssu_ref[...] + b) / pe
    su = psi * (jnp.log(1.0 + jnp.exp(-gu)) + gu)
    o2[0] = jnp.sum(su) / NUM_SURV / BATCH
    gv = (ssv_ref[...] + b) / pe
    sv = psi * (jnp.log(1.0 + jnp.exp(-gv)) + gv)
    o3[0] = jnp.sum(sv) / NUM_SURV / BATCH


def _losses(s_lam, s_su, s_sv, b_omega, psi):
    o1, o2, o3 = pl.pallas_call(
        _loss_body,
        in_specs=[
            pl.BlockSpec(memory_space=pltpu.SMEM),
            pl.BlockSpec(memory_space=pltpu.SMEM),
            pl.BlockSpec((BATCH // 128, 128), lambda: (0, 0)),
            pl.BlockSpec((BATCH * NUM_SURV // 128, 128), lambda: (0, 0)),
            pl.BlockSpec((BATCH * NUM_SURV // 128, 128), lambda: (0, 0)),
        ],
        out_specs=[
            pl.BlockSpec(memory_space=pltpu.SMEM),
            pl.BlockSpec(memory_space=pltpu.SMEM),
            pl.BlockSpec(memory_space=pltpu.SMEM),
        ],
        out_shape=[jax.ShapeDtypeStruct((1,), jnp.float32)] * 3,
    )(
        b_omega, psi,
        s_lam.reshape(BATCH // 128, 128),
        s_su.reshape(BATCH * NUM_SURV // 128, 128),
        s_sv.reshape(BATCH * NUM_SURV // 128, 128),
    )
    return o1[0], o2[0], o3[0]


def kernel(all_embeddings, assoc, src, pos_dst, neg_dst_surv, neg_src_surv,
           W_omega, b_omega, psi):
    assoc_i = assoc.astype(jnp.int32)
    p, q = _compute_pq(all_embeddings, W_omega)
    s_lam, s_su, s_sv = _sc_logits(
        assoc_i, p, q, src, pos_dst, neg_dst_surv, neg_src_surv)
    return _losses(s_lam, s_su, s_sv, b_omega, psi)


# speedup vs baseline: 59.7068x; 1.0822x over previous
import functools

import jax
import jax.numpy as jnp
from jax import lax
from jax.experimental import pallas as pl
from jax.experimental.pallas import tpu as pltpu
from jax.experimental.pallas import tpu_sc as plsc

EMBED_DIM = 512
NUM_SURV = 10
N_NODES = 10000
BATCH = 8192
_ROWS_PER_BLK = 1000


_BLK = 1024
_N_PAD = 10240


def _pq_body(w_ref, e_ref, p_ref, q_ref):
    i = pl.program_id(0)
    e = e_ref[...]
    wu = w_ref[:, :EMBED_DIM]
    wv = w_ref[:, EMBED_DIM:]
    dn = (((1,), (1,)), ((), ()))
    pt = lax.dot_general(wu, e, dn, preferred_element_type=jnp.float32)
    qt = lax.dot_general(wv, e, dn, preferred_element_type=jnp.float32)
    sl = pl.ds(i * _BLK, _BLK)
    p_ref[sl] = pt.reshape(_BLK)
    q_ref[sl] = qt.reshape(_BLK)


def _compute_pq(all_embeddings, w_omega):
    p, q = pl.pallas_call(
        _pq_body,
        grid=(_N_PAD // _BLK,),
        in_specs=[
            pl.BlockSpec((1, 2 * EMBED_DIM), lambda i: (0, 0)),
            pl.BlockSpec((_BLK, EMBED_DIM), lambda i: (i, 0)),
        ],
        out_specs=[
            pl.BlockSpec((_N_PAD,), lambda i: (0,)),
            pl.BlockSpec((_N_PAD,), lambda i: (0,)),
        ],
        out_shape=[
            jax.ShapeDtypeStruct((_N_PAD,), jnp.float32),
            jax.ShapeDtypeStruct((_N_PAD,), jnp.float32),
        ],
    )(w_omega, all_embeddings)
    return p, q


def _sc_logits(assoc, p, q, src, pos_dst, neg_dst_surv, neg_src_surv):
    mesh = plsc.VectorSubcoreMesh(core_axis_name="c", subcore_axis_name="s")
    nw = mesh.num_cores * mesh.num_subcores
    nb = BATCH // nw
    ns = (BATCH * NUM_SURV) // nw
    num_cores = mesh.num_cores
    rep_idx = jnp.arange(ns, dtype=jnp.int32) // NUM_SURV

    @functools.partial(
        pl.kernel,
        out_type=[
            jax.ShapeDtypeStruct((BATCH,), jnp.float32),
            jax.ShapeDtypeStruct((BATCH * NUM_SURV,), jnp.float32),
            jax.ShapeDtypeStruct((BATCH * NUM_SURV,), jnp.float32),
        ],
        mesh=mesh,
        compiler_params=pltpu.CompilerParams(needs_layout_passes=False),
        scratch_types=[
            pltpu.VMEM((N_NODES,), jnp.int32),
            pltpu.VMEM((_N_PAD,), jnp.float32),
            pltpu.VMEM((_N_PAD,), jnp.float32),
            pltpu.VMEM((nb,), jnp.int32),
            pltpu.VMEM((nb,), jnp.int32),
            pltpu.VMEM((ns,), jnp.int32),
            pltpu.VMEM((ns,), jnp.int32),
            pltpu.VMEM((ns,), jnp.int32),
            pltpu.VMEM((nb,), jnp.float32),
            pltpu.VMEM((nb,), jnp.float32),
            pltpu.VMEM((nb,), jnp.float32),
            pltpu.VMEM((ns,), jnp.float32),
            pltpu.VMEM((ns,), jnp.float32),
            pltpu.SemaphoreType.DMA,
            pltpu.SemaphoreType.DMA,
        ],
    )
    def k(assoc_h, p_h, q_h, src_h, pos_h, negd_h, negs_h, rep_h,
          olam_h, osu_h, osv_h,
          assoc_v, p_v, q_v, src_v, pos_v, negd_v, negs_v, rep_v,
          psrc_v, qdst_v, lam_v, su_v, sv_v, sem_in, sem_out):
        wid = lax.axis_index("s") * num_cores + lax.axis_index("c")
        copies = [
            pltpu.async_copy(src_h.at[pl.ds(wid * nb, nb)], src_v, sem_in),
            pltpu.async_copy(pos_h.at[pl.ds(wid * nb, nb)], pos_v, sem_in),
            pltpu.async_copy(negd_h.at[pl.ds(wid * ns, ns)], negd_v, sem_in),
            pltpu.async_copy(negs_h.at[pl.ds(wid * ns, ns)], negs_v, sem_in),
            pltpu.async_copy(rep_h, rep_v, sem_in),
            pltpu.async_copy(assoc_h, assoc_v, sem_in),
            pltpu.async_copy(p_h, p_v, sem_in),
            pltpu.async_copy(q_h, q_v, sem_in),
        ]
        for c in copies:
            c.wait()

        def lam_body(t, carry):
            for u in range(2):
                sl = pl.ds(t * 32 + u * 16, 16)
                pe = plsc.load_gather(
                    p_v, [plsc.load_gather(assoc_v, [src_v[sl]])])
                qe = plsc.load_gather(
                    q_v, [plsc.load_gather(assoc_v, [pos_v[sl]])])
                psrc_v[sl] = pe
                qdst_v[sl] = qe
                lam_v[sl] = pe + qe
            return carry

        lax.fori_loop(0, nb // 32, lam_body, 0)
        out_lam = pltpu.async_copy(lam_v, olam_h.at[pl.ds(wid * nb, nb)], sem_out)

        def surv_body(t, carry):
            for u in range(4):
                sl = pl.ds(t * 64 + u * 16, 16)
                ri = rep_v[sl]
                pe = plsc.load_gather(psrc_v, [ri])
                qe = plsc.load_gather(
                    q_v, [plsc.load_gather(assoc_v, [negd_v[sl]])])
                su_v[sl] = pe + qe
                pe2 = plsc.load_gather(
                    p_v, [plsc.load_gather(assoc_v, [negs_v[sl]])])
                qe2 = plsc.load_gather(qdst_v, [ri])
                sv_v[sl] = pe2 + qe2
            return carry

        lax.fori_loop(0, ns // 64, surv_body, 0)

        out_su = pltpu.async_copy(su_v, osu_h.at[pl.ds(wid * ns, ns)], sem_out)
        out_sv = pltpu.async_copy(sv_v, osv_h.at[pl.ds(wid * ns, ns)], sem_out)
        out_lam.wait()
        out_su.wait()
        out_sv.wait()

    return k(assoc, p, q, src, pos_dst, neg_dst_surv, neg_src_surv, rep_idx)


def _loss_body(b_ref, psi_ref, slam_ref, ssu_ref, ssv_ref, o1, o2, o3):
    b = b_ref[0]
    psi = psi_ref[0]
    pe = psi + 1e-7
    gl = (slam_ref[...] + b) / pe
    lam = psi * (jnp.log(1.0 + jnp.exp(-gl)) + gl)
    o1[0] = -jnp.sum(jnp.log(lam + 1e-10)) / BATCH
    gu = (ssu_ref[...] + b) / pe
    su = psi * (jnp.log(1.0 + jnp.exp(-gu)) + gu)
    o2[0] = jnp.sum(su) / NUM_SURV / BATCH
    gv = (ssv_ref[...] + b) / pe
    sv = psi * (jnp.log(1.0 + jnp.exp(-gv)) + gv)
    o3[0] = jnp.sum(sv) / NUM_SURV / BATCH


def _losses(s_lam, s_su, s_sv, b_omega, psi):
    o1, o2, o3 = pl.pallas_call(
        _loss_body,
        in_specs=[
            pl.BlockSpec(memory_space=pltpu.SMEM),
            pl.BlockSpec(memory_space=pltpu.SMEM),
            pl.BlockSpec((BATCH // 128, 128), lambda: (0, 0)),
            pl.BlockSpec((BATCH * NUM_SURV // 128, 128), lambda: (0, 0)),
            pl.BlockSpec((BATCH * NUM_SURV // 128, 128), lambda: (0, 0)),
        ],
        out_specs=[
            pl.BlockSpec(memory_space=pltpu.SMEM),
            pl.BlockSpec(memory_space=pltpu.SMEM),
            pl.BlockSpec(memory_space=pltpu.SMEM),
        ],
        out_shape=[jax.ShapeDtypeStruct((1,), jnp.float32)] * 3,
    )(
        b_omega, psi,
        s_lam.reshape(BATCH // 128, 128),
        s_su.reshape(BATCH * NUM_SURV // 128, 128),
        s_sv.reshape(BATCH * NUM_SURV // 128, 128),
    )
    return o1[0], o2[0], o3[0]


def kernel(all_embeddings, assoc, src, pos_dst, neg_dst_surv, neg_src_surv,
           W_omega, b_omega, psi):
    assoc_i = assoc.astype(jnp.int32)
    p, q = _compute_pq(all_embeddings, W_omega)
    s_lam, s_su, s_sv = _sc_logits(
        assoc_i, p, q, src, pos_dst, neg_dst_surv, neg_src_surv)
    return _losses(s_lam, s_su, s_sv, b_omega, psi)
